# baseline (speedup 1.0000x reference)
import functools
import math

import jax
import jax.numpy as jnp
from jax import lax
from jax.experimental import pallas as pl
from jax.experimental.pallas import tpu as pltpu
from jax.experimental.pallas import tpu_sc as plsc

D_MODEL = 64
SCALE = math.sqrt(D_MODEL)
LANES = 16
NUM_CORES = 2
NUM_SUBCORES = 16
NUM_WORKERS = NUM_CORES * NUM_SUBCORES
BW = 128


@functools.lru_cache(maxsize=None)
def _build(S0, S1):
    n_chunks = BW // LANES

    mesh = plsc.VectorSubcoreMesh(
        core_axis_name="c", subcore_axis_name="s",
        num_cores=NUM_CORES, num_subcores=NUM_SUBCORES)

    @functools.partial(
        pl.kernel,
        mesh=mesh,
        out_type=jax.ShapeDtypeStruct((S1, D_MODEL, S0), jnp.float32),
        scratch_types=[
            pltpu.VMEM((S1, BW), jnp.int32),
            pltpu.VMEM((BW,), jnp.int32),
            pltpu.VMEM((BW,), jnp.int32),
            pltpu.VMEM((BW, 2 * D_MODEL + 1), jnp.float32),
            pltpu.VMEM((BW, 2 * D_MODEL + 1), jnp.float32),
            pltpu.VMEM((D_MODEL, BW), jnp.float32),
            pltpu.SemaphoreType.DMA,
            pltpu.SemaphoreType.DMA,
        ],
        compiler_params=pltpu.CompilerParams(needs_layout_passes=False),
    )
    def emb(xt_hbm, tab2_hbm, out_hbm, idx_all, qa, qb, rowsa, rowsb,
            outt, sema, semb):
        wid = lax.axis_index("s") * NUM_CORES + lax.axis_index("c")
        r0 = wid * BW

        pltpu.sync_copy(xt_hbm.at[:, pl.ds(r0, BW)], idx_all)

        def fire(j, q_v, rows_v, sem):
            for c in range(n_chunks):
                sl = pl.ds(c * LANES, LANES)
                q_v[sl] = lax.shift_right_logical(idx_all[j, sl], 1)
            pltpu.async_copy(tab2_hbm.at[q_v], rows_v.at[:, pl.ds(0, 2 * D_MODEL)], sem)

        def process(j, q_v, rows_v, sem):
            pltpu.make_async_copy(
                tab2_hbm.at[q_v], rows_v.at[:, pl.ds(0, 2 * D_MODEL)], sem).wait()
            for c in range(n_chunks):
                sl = pl.ds(c * LANES, LANES)
                row_c = jnp.arange(LANES, dtype=jnp.int32) + (c * LANES)
                off_c = (idx_all[j, sl] & 1) << 6

                def kbody(k, carry, row_c=row_c, off_c=off_c, sl=sl):
                    col = off_c + k
                    v = plsc.load_gather(rows_v, [row_c, col])
                    outt[k, sl] = v * SCALE
                    return carry

                lax.fori_loop(0, D_MODEL, kbody, 0, unroll=8)
            pltpu.sync_copy(outt, out_hbm.at[j, :, pl.ds(r0, BW)])

        fire(0, qa, rowsa, sema)

        def pair_body(jj, carry):
            j0 = 2 * jj
            j1 = j0 + 1
            fire(j1, qb, rowsb, semb)
            process(j0, qa, rowsa, sema)

            @pl.when(j1 + 1 < S1)
            def _():
                fire(j1 + 1, qa, rowsa, sema)

            process(j1, qb, rowsb, semb)
            return carry

        lax.fori_loop(0, S1 // 2, pair_body, 0)

    return emb


def kernel(x, table):
    S0, S1 = x.shape
    xt = x.T.astype(jnp.int32)
    tab2 = table.reshape(table.shape[0] // 2, 2 * D_MODEL)
    out = _build(S0, S1)(xt, tab2)
    return out.transpose(2, 0, 1)


# speedup vs baseline: 1.5636x; 1.5636x over previous
import functools
import math

import jax
import jax.numpy as jnp
from jax import lax
from jax.experimental import pallas as pl
from jax.experimental.pallas import tpu as pltpu
from jax.experimental.pallas import tpu_sc as plsc

D_MODEL = 64
SCALE = math.sqrt(D_MODEL)
LANES = 16
NUM_CORES = 2
NUM_SUBCORES = 16
NUM_WORKERS = NUM_CORES * NUM_SUBCORES
TW = 2 * D_MODEL


@functools.lru_cache(maxsize=None)
def _build(S0, S1):
    B = S0 * S1
    rows_per_w = S0 // NUM_WORKERS
    npad = (-S1) % LANES
    splits = []
    off = 0
    while off < S1:
        n = min(128, S1 - off)
        splits.append((off, n))
        off += n

    mesh = plsc.VectorSubcoreMesh(
        core_axis_name="c", subcore_axis_name="s",
        num_cores=NUM_CORES, num_subcores=NUM_SUBCORES)

    @functools.partial(
        pl.kernel,
        mesh=mesh,
        out_type=jax.ShapeDtypeStruct((B, D_MODEL), jnp.float32),
        scratch_types=[
            pltpu.VMEM((rows_per_w * S1,), jnp.int32),
            pltpu.VMEM((S1, TW), jnp.float32),
            pltpu.VMEM((S1, TW), jnp.float32),
            pltpu.VMEM((S1, D_MODEL), jnp.float32),
            pltpu.SemaphoreType.DMA,
            pltpu.SemaphoreType.DMA,
        ],
    )
    def emb(x2_hbm, tabw_hbm, out_hbm, idx_all, rowsa, rowsb, outv,
            sema, semb):
        wid = lax.axis_index("s") * NUM_CORES + lax.axis_index("c")
        base = wid * rows_per_w * S1

        pltpu.sync_copy(x2_hbm.at[pl.ds(base, rows_per_w * S1)], idx_all)

        def fire(i, rows_v, sem):
            o = i * S1
            for (so, n) in splits:
                pltpu.async_copy(
                    tabw_hbm.at[idx_all.at[pl.ds(o + so, n)]],
                    rows_v.at[pl.ds(so, n)], sem)

        def process(i, rows_v, sem):
            o = i * S1
            for (so, n) in splits:
                pltpu.make_async_copy(
                    tabw_hbm.at[idx_all.at[pl.ds(o + so, n)]],
                    rows_v.at[pl.ds(so, n)], sem).wait()

            def scale_row(r, carry):
                for c in range(D_MODEL // LANES):
                    sl = pl.ds(c * LANES, LANES)
                    outv[r, sl] = rows_v[r, sl] * SCALE
                return carry

            lax.fori_loop(0, S1, scale_row, 0, unroll=4)
            pltpu.sync_copy(outv, out_hbm.at[pl.ds(base + i * S1, S1)])

        fire(0, rowsa, sema)

        def pair_body(ii, carry):
            i0 = 2 * ii
            fire(i0 + 1, rowsb, semb)
            process(i0, rowsa, sema)

            @pl.when(i0 + 2 < rows_per_w)
            def _():
                fire(i0 + 2, rowsa, sema)

            process(i0 + 1, rowsb, semb)
            return carry

        lax.fori_loop(0, rows_per_w // 2, pair_body, 0)

    return emb


def kernel(x, table):
    S0, S1 = x.shape
    x1 = x.reshape(S0 * S1).astype(jnp.int32)
    tabw = jnp.pad(table, ((0, 0), (0, TW - D_MODEL)))
    out = _build(S0, S1)(x1, tabw)
    return out.reshape(S0, S1, D_MODEL)


# speedup vs baseline: 1.7236x; 1.1023x over previous
import functools
import math

import jax
import jax.numpy as jnp
from jax import lax
from jax.experimental import pallas as pl
from jax.experimental.pallas import tpu as pltpu
from jax.experimental.pallas import tpu_sc as plsc

D_MODEL = 64
SCALE = math.sqrt(D_MODEL)
LANES = 16
NUM_CORES = 2
NUM_SUBCORES = 16
NUM_WORKERS = NUM_CORES * NUM_SUBCORES
TW = 2 * D_MODEL


@functools.lru_cache(maxsize=None)
def _build(S0, S1):
    B = S0 * S1
    rows_per_w = S0 // NUM_WORKERS
    npad = (-S1) % LANES
    splits = []
    off = 0
    while off < S1:
        n = min(128, S1 - off)
        splits.append((off, n))
        off += n

    mesh = plsc.VectorSubcoreMesh(
        core_axis_name="c", subcore_axis_name="s",
        num_cores=NUM_CORES, num_subcores=NUM_SUBCORES)

    BLK = 128
    nblk = rows_per_w * S1 // BLK
    DEPTH = 4

    @functools.partial(
        pl.kernel,
        mesh=mesh,
        out_type=jax.ShapeDtypeStruct((B, D_MODEL), jnp.float32),
        scratch_types=[
            pltpu.VMEM((rows_per_w * S1,), jnp.int32),
            [pltpu.VMEM((BLK, TW), jnp.float32) for _ in range(DEPTH)],
            [pltpu.VMEM((BLK, D_MODEL), jnp.float32) for _ in range(2)],
            [pltpu.SemaphoreType.DMA for _ in range(DEPTH)],
            [pltpu.SemaphoreType.DMA for _ in range(2)],
        ],
    )
    def emb(x1_hbm, tabw_hbm, out_hbm, idx_all, rows, outv, gsem, ssem):
        wid = lax.axis_index("s") * NUM_CORES + lax.axis_index("c")
        base = wid * rows_per_w * S1

        pltpu.sync_copy(x1_hbm.at[pl.ds(base, rows_per_w * S1)], idx_all)

        def fire(i, b):
            pltpu.async_copy(
                tabw_hbm.at[idx_all.at[pl.ds(i * BLK, BLK)]],
                rows[b], gsem[b])

        def process(i, b, ob):
            pltpu.make_async_copy(
                tabw_hbm.at[idx_all.at[pl.ds(i * BLK, BLK)]],
                rows[b], gsem[b]).wait()

            @pl.when(i >= 2)
            def _():
                pltpu.make_async_copy(
                    outv[ob], out_hbm.at[pl.ds(base, BLK)], ssem[ob]).wait()

            def scale_row(r, carry):
                for c in range(D_MODEL // LANES):
                    sl = pl.ds(c * LANES, LANES)
                    outv[ob][r, sl] = rows[b][r, sl] * SCALE
                return carry

            lax.fori_loop(0, BLK, scale_row, 0, unroll=4)
            pltpu.async_copy(
                outv[ob], out_hbm.at[pl.ds(base + i * BLK, BLK)], ssem[ob])

        for t in range(DEPTH - 1):
            fire(t, t)

        def quad_body(g, carry):
            i0 = 4 * g
            for t in range(4):
                i = i0 + t

                @pl.when(i + DEPTH - 1 < nblk)
                def _(i=i, t=t):
                    fire(i + DEPTH - 1, (t + DEPTH - 1) % DEPTH)

                process(i, t % DEPTH, t % 2)
            return carry

        lax.fori_loop(0, nblk // 4, quad_body, 0)

        for ob in range(2):
            pltpu.make_async_copy(
                outv[ob], out_hbm.at[pl.ds(base, BLK)], ssem[ob]).wait()

    return emb


def kernel(x, table):
    S0, S1 = x.shape
    x1 = x.reshape(S0 * S1).astype(jnp.int32)
    tabw = jnp.pad(table, ((0, 0), (0, TW - D_MODEL)))
    out = _build(S0, S1)(x1, tabw)
    return out.reshape(S0, S1, D_MODEL)


# speedup vs baseline: 1.7238x; 1.0002x over previous
import functools
import math

import jax
import jax.numpy as jnp
from jax import lax
from jax.experimental import pallas as pl
from jax.experimental.pallas import tpu as pltpu
from jax.experimental.pallas import tpu_sc as plsc

D_MODEL = 64
SCALE = math.sqrt(D_MODEL)
LANES = 16
NUM_CORES = 2
NUM_SUBCORES = 16
NUM_WORKERS = NUM_CORES * NUM_SUBCORES
TW = 2 * D_MODEL


@functools.lru_cache(maxsize=None)
def _build(S0, S1):
    B = S0 * S1
    rows_per_w = S0 // NUM_WORKERS

    mesh = plsc.VectorSubcoreMesh(
        core_axis_name="c", subcore_axis_name="s",
        num_cores=NUM_CORES, num_subcores=NUM_SUBCORES)

    BLK = 128
    nblk = rows_per_w * S1 // BLK
    DEPTH = 4

    @functools.partial(
        pl.kernel,
        mesh=mesh,
        out_type=jax.ShapeDtypeStruct((B, D_MODEL), jnp.float32),
        scratch_types=[
            pltpu.VMEM((rows_per_w * S1,), jnp.int32),
            [pltpu.VMEM((BLK, TW), jnp.float32) for _ in range(DEPTH)],
            [pltpu.VMEM((BLK, D_MODEL), jnp.float32) for _ in range(2)],
            [pltpu.SemaphoreType.DMA for _ in range(DEPTH)],
            [pltpu.SemaphoreType.DMA for _ in range(2)],
        ],
    )
    def emb(x1_hbm, tabw_hbm, out_hbm, idx_all, rows, outv, gsem, ssem):
        wid = lax.axis_index("s") * NUM_CORES + lax.axis_index("c")
        base = wid * rows_per_w * S1

        pltpu.sync_copy(x1_hbm.at[pl.ds(base, rows_per_w * S1)], idx_all)

        def fire(i, b):
            pltpu.async_copy(
                tabw_hbm.at[idx_all.at[pl.ds(i * BLK, BLK)]],
                rows[b], gsem[b])

        def process(i, b, ob):
            pltpu.make_async_copy(
                tabw_hbm.at[idx_all.at[pl.ds(i * BLK, BLK)]],
                rows[b], gsem[b]).wait()

            @pl.when(i >= 2)
            def _():
                pltpu.make_async_copy(
                    outv[ob], out_hbm.at[pl.ds(base, BLK)], ssem[ob]).wait()

            def scale_row(r, carry):
                for c in range(D_MODEL // LANES):
                    sl = pl.ds(c * LANES, LANES)
                    outv[ob][r, sl] = rows[b][r, sl] * SCALE
                return carry

            lax.fori_loop(0, BLK, scale_row, 0, unroll=4)
            pltpu.async_copy(
                outv[ob], out_hbm.at[pl.ds(base + i * BLK, BLK)], ssem[ob])

        for t in range(DEPTH - 1):
            fire(t, t)

        def quad_body(g, carry):
            i0 = 4 * g
            for t in range(4):
                i = i0 + t

                @pl.when(i + DEPTH - 1 < nblk)
                def _(i=i, t=t):
                    fire(i + DEPTH - 1, (t + DEPTH - 1) % DEPTH)

                process(i, t % DEPTH, t % 2)
            return carry

        lax.fori_loop(0, nblk // 4, quad_body, 0)

        for ob in range(2):
            pltpu.make_async_copy(
                outv[ob], out_hbm.at[pl.ds(base, BLK)], ssem[ob]).wait()

    return emb


def kernel(x, table):
    S0, S1 = x.shape
    x1 = x.reshape(S0 * S1).astype(jnp.int32)
    tabw = jnp.pad(table, ((0, 0), (0, TW - D_MODEL)))
    out = _build(S0, S1)(x1, tabw)
    return out.reshape(S0, S1, D_MODEL)
